# baseline (speedup 1.0000x reference)
import functools

import jax
import jax.numpy as jnp
from jax.experimental import pallas as pl
from jax.experimental.pallas import tpu as pltpu


def _crelu(x):
    return jnp.clip(x, 0.0, 1.0)


def _nnue_body(wf_ref, bf_ref, w8_ref, stm_ref, ftb8_ref, l1wT_ref, l1b_ref,
               l2wT_ref, l2b_ref, l3wT_ref, l3b_ref, out_ref):
    w8 = w8_ref[...]
    A = jnp.dot(wf_ref[...], w8, preferred_element_type=jnp.float32)
    C = jnp.dot(bf_ref[...], w8, preferred_element_type=jnp.float32)
    lane = jax.lax.broadcasted_iota(jnp.int32, A.shape, 1)
    first_half = lane < 4
    wb = jnp.where(first_half, A, C)
    bw = jnp.where(first_half, C, A)
    stm = stm_ref[...]
    acc = stm * wb + (1.0 - stm) * bw + ftb8_ref[...]
    x = _crelu(acc)
    x = _crelu(jnp.dot(x, l1wT_ref[...],
                       preferred_element_type=jnp.float32) + l1b_ref[...])
    x = _crelu(jnp.dot(x, l2wT_ref[...],
                       preferred_element_type=jnp.float32) + l2b_ref[...])
    out_ref[...] = jnp.dot(x, l3wT_ref[...],
                           preferred_element_type=jnp.float32) + l3b_ref[...]


@functools.partial(jax.jit, static_argnames=("bm",))
def _nnue(wfts, bfts, stm, ft_w, ft_b, l1_w, l1_b, l2_w, l2_b, l3_w, l3_b,
          bm=32):
    B, F = wfts.shape
    ftwT = ft_w.T
    w8 = jnp.concatenate([ftwT, ftwT], axis=1)
    ftb8 = jnp.concatenate([ft_b, ft_b]).reshape(1, 8)
    grid = (B // bm,)
    return pl.pallas_call(
        _nnue_body,
        grid=grid,
        in_specs=[
            pl.BlockSpec((bm, F), lambda i: (i, 0)),
            pl.BlockSpec((bm, F), lambda i: (i, 0)),
            pl.BlockSpec((F, 8), lambda i: (0, 0)),
            pl.BlockSpec((bm, 1), lambda i: (i, 0)),
            pl.BlockSpec((1, 8), lambda i: (0, 0)),
            pl.BlockSpec((8, 8), lambda i: (0, 0)),
            pl.BlockSpec((1, 8), lambda i: (0, 0)),
            pl.BlockSpec((8, 8), lambda i: (0, 0)),
            pl.BlockSpec((1, 8), lambda i: (0, 0)),
            pl.BlockSpec((8, 1), lambda i: (0, 0)),
            pl.BlockSpec((1, 1), lambda i: (0, 0)),
        ],
        out_specs=pl.BlockSpec((bm, 1), lambda i: (i, 0)),
        out_shape=jax.ShapeDtypeStruct((B, 1), jnp.float32),
        compiler_params=pltpu.CompilerParams(
            dimension_semantics=("parallel",),
        ),
    )(wfts, bfts, w8, stm, ftb8,
      l1_w.T, l1_b.reshape(1, 8),
      l2_w.T, l2_b.reshape(1, 8),
      l3_w.T, l3_b.reshape(1, 1))


def kernel(wfts, bfts, stm, ft_w, ft_b, l1_w, l1_b, l2_w, l2_b, l3_w, l3_b):
    return _nnue(wfts, bfts, stm, ft_w, ft_b,
                 l1_w, l1_b, l2_w, l2_b, l3_w, l3_b)
